# Initial kernel scaffold; baseline (speedup 1.0000x reference)
import jax
import jax.numpy as jnp
from jax.experimental import pallas as pl


def kernel(x, edge_index, W1_l, W1_r, b1, bn1_g, bn1_b, W2_l, W2_r, b2, bn2_g, bn2_b, W3_l, W3_r, b3):
    raise NotImplementedError("write your pallas kernel here")


# speedup vs baseline: 8.2570x; 8.2570x over previous
import functools

import jax
import jax.numpy as jnp
from jax import lax
from jax.experimental import pallas as pl
from jax.experimental.pallas import tpu as pltpu
from jax.experimental.pallas import tpu_sc as plsc

N = 100_000
E = 1_600_000
N_ACC = 102_400
ROWS_PER_TILE = 6400
ZCH = 800
NCH = ROWS_PER_TILE // ZCH
EBLK = 128
E_PAD = 1_638_400
NBLKS = E_PAD // EBLK
BPW = NBLKS // 32
KGRP = 16
NGRP = BPW // KGRP
BN = 2000
GRID = N // BN


def _seg_sum_body(src_hbm, dst_hbm, table_hbm, out_hbm,
                  srcv, dstv, rows, zbuf, acc, sem):
    cid = lax.axis_index("c")
    sid = lax.axis_index("s")
    wid = cid * 16 + sid

    def _zero_zbuf(i, _):
        zbuf[i] = jnp.zeros((16,), jnp.float32)
        return 0
    lax.fori_loop(0, ZCH, _zero_zbuf, 0)
    row0 = sid * ROWS_PER_TILE
    for z in range(NCH):
        pltpu.sync_copy(zbuf, acc.at[pl.ds(row0 + z * ZCH, ZCH)])
    plsc.subcore_barrier()

    blk0 = wid * BPW

    def _group(g, _):
        gb = blk0 + g * KGRP
        pltpu.sync_copy(src_hbm.at[pl.ds(gb, KGRP)], srcv)
        pltpu.sync_copy(dst_hbm.at[pl.ds(gb, KGRP)], dstv)
        for j in range(KGRP):
            pltpu.async_copy(table_hbm.at[srcv.at[j]], rows, sem).wait()
            pltpu.sync_copy(rows, acc.at[dstv.at[j]], add=True)
        return 0
    lax.fori_loop(0, NGRP, _group, 0)
    plsc.subcore_barrier()

    for z in range(NCH):
        r = row0 + z * ZCH
        pltpu.sync_copy(acc.at[pl.ds(r, ZCH)], zbuf)
        pltpu.sync_copy(zbuf, out_hbm.at[cid].at[pl.ds(r, ZCH)])


_seg_sum = functools.partial(
    pl.kernel,
    out_type=jax.ShapeDtypeStruct((2, N_ACC, 16), jnp.float32),
    mesh=plsc.VectorSubcoreMesh(core_axis_name="c", subcore_axis_name="s"),
    scratch_types=[
        pltpu.VMEM((KGRP, 128), jnp.int32),
        pltpu.VMEM((KGRP, 128), jnp.int32),
        pltpu.VMEM((EBLK, 16), jnp.float32),
        pltpu.VMEM((ZCH, 16), jnp.float32),
        pltpu.VMEM_SHARED((N_ACC, 16), jnp.float32),
        pltpu.SemaphoreType.DMA,
    ],
    compiler_params=pltpu.CompilerParams(use_tc_tiling_on_sc=False),
)(_seg_sum_body)


def _segment_sum16(table, src2, dst2):
    return _seg_sum(src2, dst2, table)


def _k_a(sum1, xpad, w1l, w1r, b1, z1, inv8, stats):
    i = pl.program_id(0)
    s = sum1[0] + sum1[1]
    inv = 1.0 / jnp.maximum(s[:, 15:16], 1.0)
    z = (s * inv) @ w1l[...] + xpad[...] @ w1r[...] + b1[...]
    z1[...] = z
    inv8[...] = jnp.broadcast_to(inv, (BN, 8))
    part = jnp.stack([jnp.sum(z, 0), jnp.sum(z * z, 0)])

    @pl.when(i == 0)
    def _():
        stats[...] = jnp.zeros_like(stats)
    stats[...] += part


def _k_b(z1, stats, g1, bb1, w2l, w2r, b2, p2a, p2b, q2):
    m = stats[0:1] / N
    v = stats[1:2] / N - m * m
    rs = lax.rsqrt(v + 1e-5)
    h = jnp.maximum((z1[...] - m) * rs * g1[...] + bb1[...], 0.0)
    p2 = h @ w2l[...]
    p2a[...] = p2[:, :16]
    p2b[...] = p2[:, 16:]
    q2[...] = h @ w2r[...] + b2[...]


def _k_c(sum2a, sum2b, q2, inv8, z2, stats):
    i = pl.program_id(0)
    sa = sum2a[0] + sum2a[1]
    sb = sum2b[0] + sum2b[1]
    s2 = jnp.concatenate([sa, sb], axis=1)
    z = s2 * inv8[:, :1] + q2[...]
    z2[...] = z
    part = jnp.stack([jnp.sum(z, 0), jnp.sum(z * z, 0)])

    @pl.when(i == 0)
    def _():
        stats[...] = jnp.zeros_like(stats)
    stats[...] += part


def _k_d(z2, stats, g2, bb2, w3l, w3r, b3, p3tab, q3):
    m = stats[0:1] / N
    v = stats[1:2] / N - m * m
    rs = lax.rsqrt(v + 1e-5)
    h = jnp.maximum((z2[...] - m) * rs * g2[...] + bb2[...], 0.0)
    p3tab[...] = h @ w3l[...]
    q3[...] = h @ w3r[...] + b3[...]


def _k_e(sum3, q3, inv8, out):
    s3 = sum3[0] + sum3[1]
    z = s3[:, :2] * inv8[:, :1] + q3[:, :2]
    mx = jnp.max(z, axis=1, keepdims=True)
    ez = jnp.exp(z - mx)
    out[...] = z - mx - jnp.log(jnp.sum(ez, axis=1, keepdims=True))


def _node_spec(width):
    return pl.BlockSpec((BN, width), lambda i: (i, 0))


def _sum_spec():
    return pl.BlockSpec((2, BN, 16), lambda i: (0, i, 0))


def _full_spec(shape):
    nd = len(shape)
    return pl.BlockSpec(shape, lambda i, _n=nd: (0,) * _n)


def kernel(x, edge_index, W1_l, W1_r, b1, bn1_g, bn1_b,
           W2_l, W2_r, b2, bn2_g, bn2_b, W3_l, W3_r, b3):
    f32 = jnp.float32
    pad = E_PAD - E
    src2 = jnp.concatenate(
        [edge_index[0], jnp.full((pad,), N, jnp.int32)]).reshape(NBLKS, 128)
    dst2 = jnp.concatenate(
        [edge_index[1], jnp.full((pad,), N, jnp.int32)]).reshape(NBLKS, 128)
    xpad = jnp.concatenate([x, jnp.ones((N, 1), f32)], axis=1)
    xpad1 = jnp.concatenate([xpad, jnp.zeros((1, 16), f32)], axis=0)
    w1l = jnp.concatenate([W1_l, jnp.zeros((1, 64), f32)], axis=0)
    w1r = jnp.concatenate([W1_r, jnp.zeros((1, 64), f32)], axis=0)
    w3l = jnp.concatenate([W3_l, jnp.zeros((32, 14), f32)], axis=1)
    w3r = jnp.concatenate([W3_r, jnp.zeros((32, 6), f32)], axis=1)
    b3p = jnp.concatenate([b3, jnp.zeros((6,), f32)]).reshape(1, 8)
    b1r = b1.reshape(1, 64)
    b2r = b2.reshape(1, 32)
    g1 = bn1_g.reshape(1, 64)
    bb1 = bn1_b.reshape(1, 64)
    g2 = bn2_g.reshape(1, 32)
    bb2 = bn2_b.reshape(1, 32)

    sum1 = _segment_sum16(xpad1, src2, dst2)

    z1, inv8, stats1 = pl.pallas_call(
        _k_a,
        grid=(GRID,),
        in_specs=[_sum_spec(), _node_spec(16), _full_spec((16, 64)),
                  _full_spec((16, 64)), _full_spec((1, 64))],
        out_specs=[_node_spec(64), _node_spec(8),
                   pl.BlockSpec((2, 64), lambda i: (0, 0))],
        out_shape=[jax.ShapeDtypeStruct((N, 64), f32),
                   jax.ShapeDtypeStruct((N, 8), f32),
                   jax.ShapeDtypeStruct((2, 64), f32)],
    )(sum1, xpad1, w1l, w1r, b1r)

    p2a, p2b, q2 = pl.pallas_call(
        _k_b,
        grid=(GRID,),
        in_specs=[_node_spec(64), _full_spec((2, 64)), _full_spec((1, 64)),
                  _full_spec((1, 64)), _full_spec((64, 32)),
                  _full_spec((64, 32)), _full_spec((1, 32))],
        out_specs=[_node_spec(16), _node_spec(16), _node_spec(32)],
        out_shape=[jax.ShapeDtypeStruct((N + 1, 16), f32),
                   jax.ShapeDtypeStruct((N + 1, 16), f32),
                   jax.ShapeDtypeStruct((N, 32), f32)],
    )(z1, stats1, g1, bb1, W2_l, W2_r, b2r)

    sum2a = _segment_sum16(p2a, src2, dst2)
    sum2b = _segment_sum16(p2b, src2, dst2)

    z2, stats2 = pl.pallas_call(
        _k_c,
        grid=(GRID,),
        in_specs=[_sum_spec(), _sum_spec(), _node_spec(32), _node_spec(8)],
        out_specs=[_node_spec(32), pl.BlockSpec((2, 32), lambda i: (0, 0))],
        out_shape=[jax.ShapeDtypeStruct((N, 32), f32),
                   jax.ShapeDtypeStruct((2, 32), f32)],
    )(sum2a, sum2b, q2, inv8)

    p3tab, q3 = pl.pallas_call(
        _k_d,
        grid=(GRID,),
        in_specs=[_node_spec(32), _full_spec((2, 32)), _full_spec((1, 32)),
                  _full_spec((1, 32)), _full_spec((32, 16)),
                  _full_spec((32, 8)), _full_spec((1, 8))],
        out_specs=[_node_spec(16), _node_spec(8)],
        out_shape=[jax.ShapeDtypeStruct((N + 1, 16), f32),
                   jax.ShapeDtypeStruct((N, 8), f32)],
    )(z2, stats2, g2, bb2, w3l, w3r, b3p)

    sum3 = _segment_sum16(p3tab, src2, dst2)

    out = pl.pallas_call(
        _k_e,
        grid=(GRID,),
        in_specs=[_sum_spec(), _node_spec(8), _node_spec(8)],
        out_specs=_node_spec(2),
        out_shape=jax.ShapeDtypeStruct((N, 2), f32),
    )(sum3, q3, inv8)
    return out


# speedup vs baseline: 11.4556x; 1.3874x over previous
import functools

import jax
import jax.numpy as jnp
from jax import lax
from jax.experimental import pallas as pl
from jax.experimental.pallas import tpu as pltpu
from jax.experimental.pallas import tpu_sc as plsc

N = 100_000
E = 1_600_000
N_ACC = 102_400
ROWS_PER_TILE = 6400
ZCH = 800
NCH = ROWS_PER_TILE // ZCH
EBLK = 128
E_PAD = 1_638_400
NBLKS = E_PAD // EBLK
BPW = NBLKS // 32
KGRP = 4
NGRP = BPW // KGRP
GROWS = KGRP * EBLK
BN = 2000
GRID = N // BN


def _seg_sum_body(src_hbm, dst_hbm, table_hbm, out_hbm,
                  srcv, dstv, rows2, acc, semg, sems):
    zbuf = rows2.at[pl.ds(0, ZCH)]
    cid = lax.axis_index("c")
    sid = lax.axis_index("s")
    wid = cid * 16 + sid

    def _zero_zbuf(i, _):
        zbuf[i] = jnp.zeros((16,), jnp.float32)
        return 0
    lax.fori_loop(0, ZCH, _zero_zbuf, 0)
    row0 = sid * ROWS_PER_TILE
    for z in range(NCH):
        pltpu.sync_copy(zbuf, acc.at[pl.ds(row0 + z * ZCH, ZCH)])
    plsc.subcore_barrier()

    blk0 = wid * BPW

    def _fire(g, buf):
        gb = blk0 + g * KGRP
        pltpu.sync_copy(src_hbm.at[pl.ds(gb, KGRP)], srcv.at[buf])
        pltpu.sync_copy(dst_hbm.at[pl.ds(gb, KGRP)], dstv.at[buf])
        for j in range(KGRP):
            pltpu.async_copy(table_hbm.at[srcv.at[buf].at[j]],
                             rows2.at[pl.ds(buf * GROWS + j * EBLK, EBLK)],
                             semg)

    def _drain(sem, buf):
        pltpu.make_async_copy(table_hbm.at[pl.ds(0, GROWS)],
                              rows2.at[pl.ds(buf * GROWS, GROWS)], sem).wait()

    _fire(0, 0)

    def _group(g, _):
        buf = lax.rem(g, 2)
        nbuf = 1 - buf

        @pl.when(g >= 1)
        def _():
            _drain(sems, nbuf)

        @pl.when(g + 1 < NGRP)
        def _():
            _fire(g + 1, nbuf)
        _drain(semg, buf)
        for j in range(KGRP):
            pltpu.async_copy(rows2.at[pl.ds(buf * GROWS + j * EBLK, EBLK)],
                             acc.at[dstv.at[buf].at[j]], sems, add=True)
        return 0
    lax.fori_loop(0, NGRP, _group, 0)
    _drain(sems, (NGRP - 1) % 2)
    plsc.subcore_barrier()

    for z in range(NCH):
        r = row0 + z * ZCH
        pltpu.sync_copy(acc.at[pl.ds(r, ZCH)], zbuf)
        pltpu.sync_copy(zbuf, out_hbm.at[cid].at[pl.ds(r, ZCH)])


_seg_sum = functools.partial(
    pl.kernel,
    out_type=jax.ShapeDtypeStruct((2, N_ACC, 16), jnp.float32),
    mesh=plsc.VectorSubcoreMesh(core_axis_name="c", subcore_axis_name="s"),
    scratch_types=[
        pltpu.VMEM((2, KGRP, 128), jnp.int32),
        pltpu.VMEM((2, KGRP, 128), jnp.int32),
        pltpu.VMEM((2 * GROWS, 16), jnp.float32),
        pltpu.VMEM_SHARED((N_ACC, 16), jnp.float32),
        pltpu.SemaphoreType.DMA,
        pltpu.SemaphoreType.DMA,
    ],
    compiler_params=pltpu.CompilerParams(use_tc_tiling_on_sc=False),
)(_seg_sum_body)


def _segment_sum16(table, src2, dst2):
    return _seg_sum(src2, dst2, table)


def _k_a(sum1, xpad, w1l, w1r, b1, z1, inv8, stats):
    i = pl.program_id(0)
    s = sum1[0] + sum1[1]
    inv = 1.0 / jnp.maximum(s[:, 15:16], 1.0)
    z = (s * inv) @ w1l[...] + xpad[...] @ w1r[...] + b1[...]
    z1[...] = z
    inv8[...] = jnp.broadcast_to(inv, (BN, 8))
    part = jnp.stack([jnp.sum(z, 0), jnp.sum(z * z, 0)])

    @pl.when(i == 0)
    def _():
        stats[...] = jnp.zeros_like(stats)
    stats[...] += part


def _k_b(z1, stats, g1, bb1, w2l, w2r, b2, p2a, p2b, q2):
    m = stats[0:1] / N
    v = stats[1:2] / N - m * m
    rs = lax.rsqrt(v + 1e-5)
    h = jnp.maximum((z1[...] - m) * rs * g1[...] + bb1[...], 0.0)
    p2 = h @ w2l[...]
    p2a[...] = p2[:, :16]
    p2b[...] = p2[:, 16:]
    q2[...] = h @ w2r[...] + b2[...]


def _k_c(sum2a, sum2b, q2, inv8, z2, stats):
    i = pl.program_id(0)
    sa = sum2a[0] + sum2a[1]
    sb = sum2b[0] + sum2b[1]
    s2 = jnp.concatenate([sa, sb], axis=1)
    z = s2 * inv8[:, :1] + q2[...]
    z2[...] = z
    part = jnp.stack([jnp.sum(z, 0), jnp.sum(z * z, 0)])

    @pl.when(i == 0)
    def _():
        stats[...] = jnp.zeros_like(stats)
    stats[...] += part


def _k_d(z2, stats, g2, bb2, w3l, w3r, b3, p3tab, q3):
    m = stats[0:1] / N
    v = stats[1:2] / N - m * m
    rs = lax.rsqrt(v + 1e-5)
    h = jnp.maximum((z2[...] - m) * rs * g2[...] + bb2[...], 0.0)
    p3tab[...] = h @ w3l[...]
    q3[...] = h @ w3r[...] + b3[...]


def _k_e(sum3, q3, inv8, out):
    s3 = sum3[0] + sum3[1]
    z = s3[:, :2] * inv8[:, :1] + q3[:, :2]
    mx = jnp.max(z, axis=1, keepdims=True)
    ez = jnp.exp(z - mx)
    out[...] = z - mx - jnp.log(jnp.sum(ez, axis=1, keepdims=True))


def _node_spec(width):
    return pl.BlockSpec((BN, width), lambda i: (i, 0))


def _sum_spec():
    return pl.BlockSpec((2, BN, 16), lambda i: (0, i, 0))


def _full_spec(shape):
    nd = len(shape)
    return pl.BlockSpec(shape, lambda i, _n=nd: (0,) * _n)


def kernel(x, edge_index, W1_l, W1_r, b1, bn1_g, bn1_b,
           W2_l, W2_r, b2, bn2_g, bn2_b, W3_l, W3_r, b3):
    f32 = jnp.float32
    pad = E_PAD - E
    src2 = jnp.concatenate(
        [edge_index[0], jnp.full((pad,), N, jnp.int32)]).reshape(NBLKS, 128)
    dst2 = jnp.concatenate(
        [edge_index[1], jnp.full((pad,), N, jnp.int32)]).reshape(NBLKS, 128)
    xpad = jnp.concatenate([x, jnp.ones((N, 1), f32)], axis=1)
    xpad1 = jnp.concatenate([xpad, jnp.zeros((1, 16), f32)], axis=0)
    w1l = jnp.concatenate([W1_l, jnp.zeros((1, 64), f32)], axis=0)
    w1r = jnp.concatenate([W1_r, jnp.zeros((1, 64), f32)], axis=0)
    w3l = jnp.concatenate([W3_l, jnp.zeros((32, 14), f32)], axis=1)
    w3r = jnp.concatenate([W3_r, jnp.zeros((32, 6), f32)], axis=1)
    b3p = jnp.concatenate([b3, jnp.zeros((6,), f32)]).reshape(1, 8)
    b1r = b1.reshape(1, 64)
    b2r = b2.reshape(1, 32)
    g1 = bn1_g.reshape(1, 64)
    bb1 = bn1_b.reshape(1, 64)
    g2 = bn2_g.reshape(1, 32)
    bb2 = bn2_b.reshape(1, 32)

    sum1 = _segment_sum16(xpad1, src2, dst2)

    z1, inv8, stats1 = pl.pallas_call(
        _k_a,
        grid=(GRID,),
        in_specs=[_sum_spec(), _node_spec(16), _full_spec((16, 64)),
                  _full_spec((16, 64)), _full_spec((1, 64))],
        out_specs=[_node_spec(64), _node_spec(8),
                   pl.BlockSpec((2, 64), lambda i: (0, 0))],
        out_shape=[jax.ShapeDtypeStruct((N, 64), f32),
                   jax.ShapeDtypeStruct((N, 8), f32),
                   jax.ShapeDtypeStruct((2, 64), f32)],
    )(sum1, xpad1, w1l, w1r, b1r)

    p2a, p2b, q2 = pl.pallas_call(
        _k_b,
        grid=(GRID,),
        in_specs=[_node_spec(64), _full_spec((2, 64)), _full_spec((1, 64)),
                  _full_spec((1, 64)), _full_spec((64, 32)),
                  _full_spec((64, 32)), _full_spec((1, 32))],
        out_specs=[_node_spec(16), _node_spec(16), _node_spec(32)],
        out_shape=[jax.ShapeDtypeStruct((N + 1, 16), f32),
                   jax.ShapeDtypeStruct((N + 1, 16), f32),
                   jax.ShapeDtypeStruct((N, 32), f32)],
    )(z1, stats1, g1, bb1, W2_l, W2_r, b2r)

    sum2a = _segment_sum16(p2a, src2, dst2)
    sum2b = _segment_sum16(p2b, src2, dst2)

    z2, stats2 = pl.pallas_call(
        _k_c,
        grid=(GRID,),
        in_specs=[_sum_spec(), _sum_spec(), _node_spec(32), _node_spec(8)],
        out_specs=[_node_spec(32), pl.BlockSpec((2, 32), lambda i: (0, 0))],
        out_shape=[jax.ShapeDtypeStruct((N, 32), f32),
                   jax.ShapeDtypeStruct((2, 32), f32)],
    )(sum2a, sum2b, q2, inv8)

    p3tab, q3 = pl.pallas_call(
        _k_d,
        grid=(GRID,),
        in_specs=[_node_spec(32), _full_spec((2, 32)), _full_spec((1, 32)),
                  _full_spec((1, 32)), _full_spec((32, 16)),
                  _full_spec((32, 8)), _full_spec((1, 8))],
        out_specs=[_node_spec(16), _node_spec(8)],
        out_shape=[jax.ShapeDtypeStruct((N + 1, 16), f32),
                   jax.ShapeDtypeStruct((N, 8), f32)],
    )(z2, stats2, g2, bb2, w3l, w3r, b3p)

    sum3 = _segment_sum16(p3tab, src2, dst2)

    out = pl.pallas_call(
        _k_e,
        grid=(GRID,),
        in_specs=[_sum_spec(), _node_spec(8), _node_spec(8)],
        out_specs=_node_spec(2),
        out_shape=jax.ShapeDtypeStruct((N, 2), f32),
    )(sum3, q3, inv8)
    return out
